# baseline (speedup 1.0000x reference)
import jax
import jax.numpy as jnp
from jax.experimental import pallas as pl


_S_BLK = 1024


def _add_body(x_ref, pe_ref, o_ref):
    o_ref[...] = x_ref[...] + pe_ref[...][None, :, :]


def kernel(x, pe):
    B, S, D = x.shape
    grid = (S // _S_BLK, B)
    return pl.pallas_call(
        _add_body,
        grid=grid,
        in_specs=[
            pl.BlockSpec((1, _S_BLK, D), lambda j, b: (b, j, 0)),
            pl.BlockSpec((_S_BLK, D), lambda j, b: (j, 0)),
        ],
        out_specs=pl.BlockSpec((1, _S_BLK, D), lambda j, b: (b, j, 0)),
        out_shape=jax.ShapeDtypeStruct((B, S, D), x.dtype),
    )(x, pe)
